# Initial kernel scaffold; baseline (speedup 1.0000x reference)
import jax
import jax.numpy as jnp
from jax.experimental import pallas as pl


def kernel(data, edge_index, edge_weight, W1, b1, W2, b2, temp):
    raise NotImplementedError("write your pallas kernel here")


# speedup vs baseline: 4.0333x; 4.0333x over previous
import math
import functools
import numpy as np
import jax
import jax.numpy as jnp
from jax import lax
from jax.experimental import pallas as pl
from jax.experimental.pallas import tpu as pltpu, tpu_sc as plsc

_N = 10000
_NP = 10240
_E = 160000
_NW = 32
_CHUNK = 5008
_EP = _NW * _CHUNK
_GROUPS = _CHUNK // 16
_K = 10
_FP = 64


def _cheby(i, x):
    if i == 0:
        return 1.0
    if i == 1:
        return x
    t0, t1 = 1.0, x
    for _ in range(2, i + 1):
        t0, t1 = t1, 2 * x * t1 - t0
    return t1


_xs = [math.cos((_K - j + 0.5) * math.pi / (_K + 1)) for j in range(_K + 1)]
_CHEBP = np.zeros((16, 16), dtype=np.float32)
for _i in range(_K + 1):
    for _j in range(_K + 1):
        _CHEBP[_i, _j] = _cheby(_i, _xs[_j])

_mesh = plsc.VectorSubcoreMesh(core_axis_name="c", subcore_axis_name="s")
_sc_params = pltpu.CompilerParams(needs_layout_passes=False)


@functools.partial(
    pl.kernel,
    out_type=jax.ShapeDtypeStruct((_NW, _N), jnp.float32),
    mesh=_mesh,
    compiler_params=_sc_params,
    scratch_types=[
        pltpu.VMEM((_N,), jnp.float32),
        pltpu.VMEM((_CHUNK,), jnp.int32),
        pltpu.VMEM((_CHUNK,), jnp.float32),
    ],
)
def _sc_deg(row_hbm, ew_hbm, deg_all_hbm, deg_v, row_v, ew_v):
    c = lax.axis_index("c")
    s = lax.axis_index("s")
    wid = s * 2 + c

    def zbody(j, carry):
        deg_v[pl.ds(j * 16, 16)] = jnp.zeros((16,), jnp.float32)
        return carry

    lax.fori_loop(0, _N // 16, zbody, 0)
    pltpu.sync_copy(row_hbm.at[pl.ds(wid * _CHUNK, _CHUNK)], row_v)
    pltpu.sync_copy(ew_hbm.at[pl.ds(wid * _CHUNK, _CHUNK)], ew_v)

    def body(g, carry):
        r16 = row_v[pl.ds(g * 16, 16)]
        e16 = ew_v[pl.ds(g * 16, 16)]
        plsc.addupdate_scatter(deg_v, [r16], e16)
        return carry

    lax.fori_loop(0, _GROUPS, body, 0)
    pltpu.sync_copy(deg_v, deg_all_hbm.at[wid])


def _tc_dis_body(deg_ref, dis_ref):
    deg = jnp.sum(deg_ref[...], axis=0)
    safe = jnp.where(deg > 0, deg, 1.0)
    dis_ref[...] = jnp.where(deg > 0, lax.rsqrt(safe), 0.0)


def _tc_dis(deg_all_r):
    return pl.pallas_call(
        _tc_dis_body,
        out_shape=jax.ShapeDtypeStruct((8, _N // 8), jnp.float32),
    )(deg_all_r)


def _tc_mlp_body(d_ref, w1_ref, b1_ref, w2_ref, b2_ref, o_ref):
    h = jnp.maximum(
        jnp.dot(w1_ref[...], d_ref[...], preferred_element_type=jnp.float32, precision=lax.Precision.HIGHEST)
        + b1_ref[...],
        0.0,
    )
    o_ref[...] = (
        jnp.dot(w2_ref[...], h, preferred_element_type=jnp.float32, precision=lax.Precision.HIGHEST) + b2_ref[...]
    )


def _tc_mlp(dataT, W1, b1c, W2p, b2p):
    blk = 1024
    grid = _NP // blk
    return pl.pallas_call(
        _tc_mlp_body,
        grid=(grid,),
        in_specs=[
            pl.BlockSpec((256, blk), lambda j: (0, j)),
            pl.BlockSpec((64, 256), lambda j: (0, 0)),
            pl.BlockSpec((64, 1), lambda j: (0, 0)),
            pl.BlockSpec((64, 64), lambda j: (0, 0)),
            pl.BlockSpec((64, 1), lambda j: (0, 0)),
        ],
        out_specs=pl.BlockSpec((64, blk), lambda j: (0, j)),
        out_shape=jax.ShapeDtypeStruct((_FP, _NP), jnp.float32),
    )(dataT, W1, b1c, W2p, b2p)


@functools.partial(
    pl.kernel,
    out_type=jax.ShapeDtypeStruct((_EP,), jnp.float32),
    mesh=_mesh,
    compiler_params=_sc_params,
    scratch_types=[
        pltpu.VMEM((_N,), jnp.float32),
        pltpu.VMEM((_CHUNK,), jnp.int32),
        pltpu.VMEM((_CHUNK,), jnp.int32),
        pltpu.VMEM((_CHUNK,), jnp.float32),
        pltpu.VMEM((_CHUNK,), jnp.float32),
    ],
)
def _sc_wnorm(row_hbm, col_hbm, ew_hbm, dis_hbm, w_hbm,
              dis_v, row_v, col_v, ew_v, w_v):
    c = lax.axis_index("c")
    s = lax.axis_index("s")
    wid = s * 2 + c
    pltpu.sync_copy(dis_hbm, dis_v)
    base = wid * _CHUNK
    pltpu.sync_copy(row_hbm.at[pl.ds(base, _CHUNK)], row_v)
    pltpu.sync_copy(col_hbm.at[pl.ds(base, _CHUNK)], col_v)
    pltpu.sync_copy(ew_hbm.at[pl.ds(base, _CHUNK)], ew_v)

    def body(g, carry):
        off = g * 16
        r16 = row_v[pl.ds(off, 16)]
        c16 = col_v[pl.ds(off, 16)]
        e16 = ew_v[pl.ds(off, 16)]
        a = plsc.load_gather(dis_v, [r16])
        b = plsc.load_gather(dis_v, [c16])
        w_v[pl.ds(off, 16)] = -(a * e16 * b)
        return carry

    lax.fori_loop(0, _GROUPS, body, 0)
    pltpu.sync_copy(w_v, w_hbm.at[pl.ds(base, _CHUNK)])


@functools.partial(
    pl.kernel,
    out_type=jax.ShapeDtypeStruct((_FP, _NP), jnp.float32),
    mesh=_mesh,
    compiler_params=_sc_params,
    scratch_types=[
        pltpu.VMEM((_NP,), jnp.float32),
        pltpu.VMEM((_NP,), jnp.float32),
        pltpu.VMEM((_NP,), jnp.float32),
        pltpu.VMEM((_NP,), jnp.float32),
        pltpu.VMEM((_NP,), jnp.float32),
        pltpu.VMEM((_NP,), jnp.float32),
        pltpu.VMEM((_NP,), jnp.float32),
        pltpu.VMEM((_NP,), jnp.float32),
        pltpu.VMEM((_CHUNK,), jnp.int32),
        pltpu.VMEM((_CHUNK,), jnp.int32),
        pltpu.VMEM((_CHUNK,), jnp.float32),
        pltpu.VMEM((16, 16), jnp.float32),
    ],
)
def _sc_cheb(xT_hbm, src_hbm, dst_hbm, w_hbm, coe_hbm, outT_hbm,
             a0, a1, y0, y1, b0, b1, o0, o1, src_v, dst_v, w_v, coe_v):
    c = lax.axis_index("c")
    s = lax.axis_index("s")
    wid = s * 2 + c
    f0 = 2 * wid

    pltpu.sync_copy(coe_hbm, coe_v)
    pltpu.sync_copy(xT_hbm.at[f0], a0)
    pltpu.sync_copy(xT_hbm.at[f0 + 1], a1)

    def zero(t0, t1):
        def zb(j, carry):
            t0[pl.ds(j * 16, 16)] = jnp.zeros((16,), jnp.float32)
            t1[pl.ds(j * 16, 16)] = jnp.zeros((16,), jnp.float32)
            return carry
        lax.fori_loop(0, _NP // 16, zb, 0)

    def edge_pass(c0, c1, t0, t1):
        zero(t0, t1)

        def chunk_body(ch, carry):
            base = ch * _CHUNK
            pltpu.sync_copy(src_hbm.at[pl.ds(base, _CHUNK)], src_v)
            pltpu.sync_copy(dst_hbm.at[pl.ds(base, _CHUNK)], dst_v)
            pltpu.sync_copy(w_hbm.at[pl.ds(base, _CHUNK)], w_v)

            def g_body(g, carry2):
                off = g * 16
                s16 = src_v[pl.ds(off, 16)]
                d16 = dst_v[pl.ds(off, 16)]
                w16 = w_v[pl.ds(off, 16)]
                v0 = plsc.load_gather(c0, [s16]) * w16
                plsc.addupdate_scatter(t0, [d16], v0)
                v1 = plsc.load_gather(c1, [s16]) * w16
                plsc.addupdate_scatter(t1, [d16], v1)
                return carry2

            lax.fori_loop(0, _GROUPS, g_body, 0)
            return carry

        lax.fori_loop(0, _NW, chunk_body, 0)

    edge_pass(a0, a1, y0, y1)

    c0v = coe_v[0]
    c1v = coe_v[1]

    def init_out(j, carry):
        ds = pl.ds(j * 16, 16)
        o0[ds] = c0v * a0[ds] + c1v * y0[ds]
        o1[ds] = c0v * a1[ds] + c1v * y1[ds]
        return carry

    lax.fori_loop(0, _NP // 16, init_out, 0)

    cur0, cur1 = y0, y1
    prev0, prev1 = a0, a1
    for i in range(2, _K + 1):
        edge_pass(cur0, cur1, b0, b1)
        civ = coe_v[i]

        def upd(j, carry):
            ds = pl.ds(j * 16, 16)
            t0 = 2.0 * b0[ds] - prev0[ds]
            prev0[ds] = t0
            o0[ds] = o0[ds] + civ * t0
            t1 = 2.0 * b1[ds] - prev1[ds]
            prev1[ds] = t1
            o1[ds] = o1[ds] + civ * t1
            return carry

        lax.fori_loop(0, _NP // 16, upd, 0)
        cur0, prev0 = prev0, cur0
        cur1, prev1 = prev1, cur1

    pltpu.sync_copy(o0, outT_hbm.at[f0])
    pltpu.sync_copy(o1, outT_hbm.at[f0 + 1])


def kernel(data, edge_index, edge_weight, W1, b1, W2, b2, temp):
    row = edge_index[0]
    col = edge_index[1]
    pad_e = _EP - _E
    row_p = jnp.concatenate([row, jnp.zeros((pad_e,), row.dtype)])
    col_p = jnp.concatenate([col, jnp.zeros((pad_e,), col.dtype)])
    ew_p = jnp.concatenate([edge_weight, jnp.zeros((pad_e,), edge_weight.dtype)])

    dataT = jnp.pad(data.T, ((0, 0), (0, _NP - _N)))
    b1c = b1[:, None]
    W2p = jnp.pad(W2, ((0, _FP - W2.shape[0]), (0, 0)))
    b2p = jnp.pad(b2, (0, _FP - b2.shape[0]))[:, None]
    coe = (2.0 / (_K + 1)) * (jnp.asarray(_CHEBP[:_K + 1, :_K + 1]) @ jax.nn.relu(temp))
    coe = coe * jnp.where(jnp.arange(_K + 1) == 0, 0.5, 1.0)
    coe_b = jnp.broadcast_to(jnp.pad(coe, (0, 5))[:, None], (16, 16))

    deg_all = _sc_deg(row_p, ew_p)
    dis2 = _tc_dis(deg_all.reshape(_NW, 8, _N // 8))
    dis = dis2.reshape(_N)
    xT = _tc_mlp(dataT, W1, b1c, W2p, b2p)
    w_neg = _sc_wnorm(row_p, col_p, ew_p, dis)
    outT = _sc_cheb(xT, row_p, col_p, w_neg, coe_b)
    return outT[:40, :_N].T


# speedup vs baseline: 5.6316x; 1.3963x over previous
import math
import functools
import numpy as np
import jax
import jax.numpy as jnp
from jax import lax
from jax.experimental import pallas as pl
from jax.experimental.pallas import tpu as pltpu, tpu_sc as plsc

_N = 10000
_NP = 10240
_E = 160000
_NW = 32
_CHUNK = 5008
_EP = _NW * _CHUNK
_GROUPS = _CHUNK // 16
_K = 10
_FP = 64


def _cheby(i, x):
    if i == 0:
        return 1.0
    if i == 1:
        return x
    t0, t1 = 1.0, x
    for _ in range(2, i + 1):
        t0, t1 = t1, 2 * x * t1 - t0
    return t1


_xs = [math.cos((_K - j + 0.5) * math.pi / (_K + 1)) for j in range(_K + 1)]
_CHEBP = np.zeros((16, 16), dtype=np.float32)
for _i in range(_K + 1):
    for _j in range(_K + 1):
        _CHEBP[_i, _j] = _cheby(_i, _xs[_j])

_mesh = plsc.VectorSubcoreMesh(core_axis_name="c", subcore_axis_name="s")
_sc_params = pltpu.CompilerParams(needs_layout_passes=False)


@functools.partial(
    pl.kernel,
    out_type=jax.ShapeDtypeStruct((_NW, _N), jnp.float32),
    mesh=_mesh,
    compiler_params=_sc_params,
    scratch_types=[
        pltpu.VMEM((_N,), jnp.float32),
        pltpu.VMEM((_CHUNK,), jnp.int32),
        pltpu.VMEM((_CHUNK,), jnp.float32),
    ],
)
def _sc_deg(row_hbm, ew_hbm, deg_all_hbm, deg_v, row_v, ew_v):
    c = lax.axis_index("c")
    s = lax.axis_index("s")
    wid = s * 2 + c

    def zbody(j, carry):
        deg_v[pl.ds(j * 16, 16)] = jnp.zeros((16,), jnp.float32)
        return carry

    lax.fori_loop(0, _N // 16, zbody, 0)
    pltpu.sync_copy(row_hbm.at[pl.ds(wid * _CHUNK, _CHUNK)], row_v)
    pltpu.sync_copy(ew_hbm.at[pl.ds(wid * _CHUNK, _CHUNK)], ew_v)

    def body(g, carry):
        r16 = row_v[pl.ds(g * 16, 16)]
        e16 = ew_v[pl.ds(g * 16, 16)]
        plsc.addupdate_scatter(deg_v, [r16], e16)
        return carry

    lax.fori_loop(0, _GROUPS, body, 0)
    pltpu.sync_copy(deg_v, deg_all_hbm.at[wid])


def _tc_dis_body(deg_ref, dis_ref):
    deg = jnp.sum(deg_ref[...], axis=0)
    safe = jnp.where(deg > 0, deg, 1.0)
    dis_ref[...] = jnp.where(deg > 0, lax.rsqrt(safe), 0.0)


def _tc_dis(deg_all_r):
    return pl.pallas_call(
        _tc_dis_body,
        out_shape=jax.ShapeDtypeStruct((8, _N // 8), jnp.float32),
    )(deg_all_r)


def _tc_mlp_body(d_ref, w1_ref, b1_ref, w2_ref, b2_ref, o_ref):
    h = jnp.maximum(
        jnp.dot(w1_ref[...], d_ref[...], preferred_element_type=jnp.float32, precision=lax.Precision.HIGHEST)
        + b1_ref[...],
        0.0,
    )
    o_ref[...] = (
        jnp.dot(w2_ref[...], h, preferred_element_type=jnp.float32, precision=lax.Precision.HIGHEST) + b2_ref[...]
    )


def _tc_mlp(dataT, W1, b1c, W2p, b2p):
    blk = 1024
    grid = _NP // blk
    return pl.pallas_call(
        _tc_mlp_body,
        grid=(grid,),
        in_specs=[
            pl.BlockSpec((256, blk), lambda j: (0, j)),
            pl.BlockSpec((64, 256), lambda j: (0, 0)),
            pl.BlockSpec((64, 1), lambda j: (0, 0)),
            pl.BlockSpec((64, 64), lambda j: (0, 0)),
            pl.BlockSpec((64, 1), lambda j: (0, 0)),
        ],
        out_specs=pl.BlockSpec((64, blk), lambda j: (0, j)),
        out_shape=jax.ShapeDtypeStruct((_FP, _NP), jnp.float32),
    )(dataT, W1, b1c, W2p, b2p)


@functools.partial(
    pl.kernel,
    out_type=jax.ShapeDtypeStruct((_EP,), jnp.float32),
    mesh=_mesh,
    compiler_params=_sc_params,
    scratch_types=[
        pltpu.VMEM((_N,), jnp.float32),
        pltpu.VMEM((_CHUNK,), jnp.int32),
        pltpu.VMEM((_CHUNK,), jnp.int32),
        pltpu.VMEM((_CHUNK,), jnp.float32),
        pltpu.VMEM((_CHUNK,), jnp.float32),
    ],
)
def _sc_wnorm(row_hbm, col_hbm, ew_hbm, dis_hbm, w_hbm,
              dis_v, row_v, col_v, ew_v, w_v):
    c = lax.axis_index("c")
    s = lax.axis_index("s")
    wid = s * 2 + c
    pltpu.sync_copy(dis_hbm, dis_v)
    base = wid * _CHUNK
    pltpu.sync_copy(row_hbm.at[pl.ds(base, _CHUNK)], row_v)
    pltpu.sync_copy(col_hbm.at[pl.ds(base, _CHUNK)], col_v)
    pltpu.sync_copy(ew_hbm.at[pl.ds(base, _CHUNK)], ew_v)

    def body(g, carry):
        off = g * 16
        r16 = row_v[pl.ds(off, 16)]
        c16 = col_v[pl.ds(off, 16)]
        e16 = ew_v[pl.ds(off, 16)]
        a = plsc.load_gather(dis_v, [r16])
        b = plsc.load_gather(dis_v, [c16])
        w_v[pl.ds(off, 16)] = -(a * e16 * b)
        return carry

    lax.fori_loop(0, _GROUPS, body, 0)
    pltpu.sync_copy(w_v, w_hbm.at[pl.ds(base, _CHUNK)])


@functools.partial(
    pl.kernel,
    out_type=jax.ShapeDtypeStruct((_FP, _NP), jnp.float32),
    mesh=_mesh,
    compiler_params=_sc_params,
    scratch_types=[
        pltpu.VMEM((_NP,), jnp.float32),
        pltpu.VMEM((_NP,), jnp.float32),
        pltpu.VMEM((_NP,), jnp.float32),
        pltpu.VMEM((_NP,), jnp.float32),
        pltpu.VMEM((_NP,), jnp.float32),
        pltpu.VMEM((_NP,), jnp.float32),
        pltpu.VMEM((_NP,), jnp.float32),
        pltpu.VMEM((_NP,), jnp.float32),
        pltpu.VMEM((_CHUNK,), jnp.int32),
        pltpu.VMEM((_CHUNK,), jnp.int32),
        pltpu.VMEM((_CHUNK,), jnp.float32),
        pltpu.VMEM((_CHUNK,), jnp.int32),
        pltpu.VMEM((_CHUNK,), jnp.int32),
        pltpu.VMEM((_CHUNK,), jnp.float32),
        pltpu.VMEM((16, 16), jnp.float32),
        pltpu.SemaphoreType.DMA,
        pltpu.SemaphoreType.DMA,
    ],
)
def _sc_cheb(xT_hbm, src_hbm, dst_hbm, w_hbm, coe_hbm, outT_hbm,
             a0, a1, y0, y1, b0, b1, o0, o1,
             src_va, dst_va, w_va, src_vb, dst_vb, w_vb, coe_v,
             sem_a, sem_b):
    c = lax.axis_index("c")
    s = lax.axis_index("s")
    wid = s * 2 + c
    f0 = 2 * wid

    pltpu.sync_copy(coe_hbm, coe_v)
    pltpu.sync_copy(xT_hbm.at[f0], a0)
    pltpu.sync_copy(xT_hbm.at[f0 + 1], a1)

    def zero(t0, t1):
        def zb(j, carry):
            t0[pl.ds(j * 16, 16)] = jnp.zeros((16,), jnp.float32)
            t1[pl.ds(j * 16, 16)] = jnp.zeros((16,), jnp.float32)
            return carry
        lax.fori_loop(0, _NP // 16, zb, 0)

    def start(ch, sv, dv, wv, sem):
        base = ch * _CHUNK
        pltpu.async_copy(src_hbm.at[pl.ds(base, _CHUNK)], sv, sem)
        pltpu.async_copy(dst_hbm.at[pl.ds(base, _CHUNK)], dv, sem)
        pltpu.async_copy(w_hbm.at[pl.ds(base, _CHUNK)], wv, sem)

    def drain(sv, dv, wv, sem):
        pltpu.make_async_copy(src_hbm.at[pl.ds(0, _CHUNK)], sv, sem).wait()
        pltpu.make_async_copy(dst_hbm.at[pl.ds(0, _CHUNK)], dv, sem).wait()
        pltpu.make_async_copy(w_hbm.at[pl.ds(0, _CHUNK)], wv, sem).wait()

    def edge_pass(c0, c1, t0, t1):
        zero(t0, t1)
        start(0, src_va, dst_va, w_va, sem_a)

        def process(sv, dv, wv):
            def g_body(g, carry2):
                off = g * 16
                s16 = sv[pl.ds(off, 16)]
                d16 = dv[pl.ds(off, 16)]
                w16 = wv[pl.ds(off, 16)]
                v0 = plsc.load_gather(c0, [s16]) * w16
                plsc.addupdate_scatter(t0, [d16], v0)
                v1 = plsc.load_gather(c1, [s16]) * w16
                plsc.addupdate_scatter(t1, [d16], v1)
                return carry2

            lax.fori_loop(0, _GROUPS, g_body, 0)

        def chunk_body(i, carry):
            start(2 * i + 1, src_vb, dst_vb, w_vb, sem_b)
            drain(src_va, dst_va, w_va, sem_a)
            process(src_va, dst_va, w_va)

            @pl.when(i < _NW // 2 - 1)
            def _():
                start(2 * i + 2, src_va, dst_va, w_va, sem_a)

            drain(src_vb, dst_vb, w_vb, sem_b)
            process(src_vb, dst_vb, w_vb)
            return carry

        lax.fori_loop(0, _NW // 2, chunk_body, 0)

    edge_pass(a0, a1, y0, y1)

    c0v = coe_v[0]
    c1v = coe_v[1]

    def init_out(j, carry):
        ds = pl.ds(j * 16, 16)
        o0[ds] = c0v * a0[ds] + c1v * y0[ds]
        o1[ds] = c0v * a1[ds] + c1v * y1[ds]
        return carry

    lax.fori_loop(0, _NP // 16, init_out, 0)

    cur0, cur1 = y0, y1
    prev0, prev1 = a0, a1
    for i in range(2, _K + 1):
        edge_pass(cur0, cur1, b0, b1)
        civ = coe_v[i]

        def upd(j, carry):
            ds = pl.ds(j * 16, 16)
            t0 = 2.0 * b0[ds] - prev0[ds]
            prev0[ds] = t0
            o0[ds] = o0[ds] + civ * t0
            t1 = 2.0 * b1[ds] - prev1[ds]
            prev1[ds] = t1
            o1[ds] = o1[ds] + civ * t1
            return carry

        lax.fori_loop(0, _NP // 16, upd, 0)
        cur0, prev0 = prev0, cur0
        cur1, prev1 = prev1, cur1

    pltpu.sync_copy(o0, outT_hbm.at[f0])
    pltpu.sync_copy(o1, outT_hbm.at[f0 + 1])


def kernel(data, edge_index, edge_weight, W1, b1, W2, b2, temp):
    row = edge_index[0]
    col = edge_index[1]
    pad_e = _EP - _E
    row_p = jnp.concatenate([row, jnp.zeros((pad_e,), row.dtype)])
    col_p = jnp.concatenate([col, jnp.zeros((pad_e,), col.dtype)])
    ew_p = jnp.concatenate([edge_weight, jnp.zeros((pad_e,), edge_weight.dtype)])

    dataT = jnp.pad(data.T, ((0, 0), (0, _NP - _N)))
    b1c = b1[:, None]
    W2p = jnp.pad(W2, ((0, _FP - W2.shape[0]), (0, 0)))
    b2p = jnp.pad(b2, (0, _FP - b2.shape[0]))[:, None]
    coe = (2.0 / (_K + 1)) * (jnp.asarray(_CHEBP[:_K + 1, :_K + 1]) @ jax.nn.relu(temp))
    coe = coe * jnp.where(jnp.arange(_K + 1) == 0, 0.5, 1.0)
    coe_b = jnp.broadcast_to(jnp.pad(coe, (0, 5))[:, None], (16, 16))

    deg_all = _sc_deg(row_p, ew_p)
    dis2 = _tc_dis(deg_all.reshape(_NW, 8, _N // 8))
    dis = dis2.reshape(_N)
    xT = _tc_mlp(dataT, W1, b1c, W2p, b2p)
    w_neg = _sc_wnorm(row_p, col_p, ew_p, dis)
    outT = _sc_cheb(xT, row_p, col_p, w_neg, coe_b)
    return outT[:40, :_N].T
